# baseline (speedup 1.0000x reference)
import functools

import jax
import jax.numpy as jnp
from jax import lax
from jax.experimental import pallas as pl
from jax.experimental.pallas import tpu as pltpu
from jax.experimental.pallas import tpu_sc as plsc

_NC = 2
_NS = 16
_C = 125
_RING = 2
_GC = 4
_ES = 5
_ZR = 16


def _neighbor_partials(eidx, x, npad):
    n, d = npad, x.shape[1]
    ngrp = eidx.shape[2]
    rpt = n // _NS
    nouter = ngrp // _ES

    mesh = plsc.VectorSubcoreMesh(core_axis_name="c", subcore_axis_name="s")

    @functools.partial(
        pl.kernel,
        out_type=jax.ShapeDtypeStruct((_NC, n, d), jnp.float32),
        mesh=mesh,
        scratch_types=[
            [pltpu.VMEM((2, _GC, _C), jnp.int32) for _ in range(_ES)],
            [pltpu.VMEM((_C, d), jnp.float32) for _ in range(_RING)],
            pltpu.VMEM((_ZR, d), jnp.float32),
            pltpu.VMEM_SHARED((n, d), jnp.float32),
            [pltpu.SemaphoreType.DMA for _ in range(_ES)],
            [pltpu.SemaphoreType.DMA for _ in range(_RING)],
            [pltpu.SemaphoreType.DMA for _ in range(_RING)],
            pltpu.SemaphoreType.DMA,
        ],
    )
    def scatter_k(edge_hbm, x_hbm, part_hbm, ebufs, rows, zero_v, acc_sh,
                  esem, gsem, ssem, zsem):
        cid = lax.axis_index("c")
        sid = lax.axis_index("s")
        w = cid * _NS + sid

        def _zrow(i, carry):
            for c16 in range(d // 16):
                zero_v[i, pl.ds(c16 * 16, 16)] = jnp.zeros((16,), jnp.float32)
            return carry

        lax.fori_loop(0, _ZR, _zrow, 0)
        nz = rpt // _ZR
        for k in range(nz):
            pltpu.async_copy(zero_v, acc_sh.at[pl.ds(sid * rpt + k * _ZR, _ZR)],
                             zsem)
        for k in range(nz):
            pltpu.make_async_copy(
                zero_v, acc_sh.at[pl.ds(sid * rpt, _ZR)], zsem).wait()
        plsc.subcore_barrier()

        def issue_e(cg, t):
            pltpu.async_copy(edge_hbm.at[0, w, cg], ebufs[t].at[0], esem[t])
            pltpu.async_copy(edge_hbm.at[1, w, cg], ebufs[t].at[1], esem[t])

        def wait_e(t):
            pltpu.make_async_copy(edge_hbm.at[0, w, 0], ebufs[t].at[0],
                                  esem[t]).wait()
            pltpu.make_async_copy(edge_hbm.at[1, w, 0], ebufs[t].at[1],
                                  esem[t]).wait()

        def issue_g(t, i, b):
            pltpu.async_copy(x_hbm.at[ebufs[t].at[0, i]], rows[b], gsem[b])

        def wait_g(t, i, b):
            pltpu.make_async_copy(x_hbm.at[ebufs[t].at[0, i]], rows[b],
                                  gsem[b]).wait()

        def issue_s(t, i, b):
            pltpu.async_copy(rows[b], acc_sh.at[ebufs[t].at[1, i]], ssem[b],
                             add=True)

        def wait_s(t, i, b):
            pltpu.make_async_copy(rows[b], acc_sh.at[ebufs[t].at[1, i]],
                                  ssem[b]).wait()

        for t in range(_ES):
            issue_e(t, t)

        def _outer(kk, carry):
            for go in range(_ES):
                for i in range(_GC):
                    b = i % _RING
                    bp = (i - 1) % _RING
                    pgo, pi = (go, i - 1) if i else ((go - 1) % _ES, _GC - 1)
                    if go == 0 and i < _RING:
                        @pl.when(kk > 0)
                        def _ws():
                            wait_s(0, 0, b)
                    else:
                        wait_s(0, 0, b)
                    if i == 0:
                        wait_e(go)
                    issue_g(go, i, b)
                    if go == 0 and i == 0:
                        @pl.when(kk > 0)
                        def _sc():
                            wait_g(0, 0, bp)
                            issue_s(pgo, pi, bp)
                    else:
                        wait_g(0, 0, bp)
                        issue_s(pgo, pi, bp)
                    if i == _RING - 1:
                        gnext = kk * _ES + go + _ES - 1
                        if go == 0:
                            pred = (kk > 0) & (gnext < ngrp)
                        else:
                            pred = gnext < ngrp
                        @pl.when(pred)
                        def _pf():
                            issue_e(gnext, (go - 1) % _ES)
            return carry

        lax.fori_loop(0, nouter, _outer, 0)
        bl = (_GC - 1) % _RING
        wait_g(0, 0, bl)
        issue_s(_ES - 1, _GC - 1, bl)
        for b in range(_RING):
            wait_s(0, 0, b)
        plsc.subcore_barrier()

        pltpu.sync_copy(acc_sh.at[pl.ds(sid * rpt, rpt)],
                        part_hbm.at[cid, pl.ds(sid * rpt, rpt)])

    return scatter_k(eidx, x)


def kernel(x, edge_index, W_neigh, b_neigh, W_self, b_self):
    n, d = x.shape
    d_out = W_neigh.shape[0]
    e = edge_index.shape[1]
    nw = _NC * _NS
    epw = e // nw
    nch = epw // _C
    ngrp = nch // _GC
    npad = -(-n // (_NS * 128)) * (_NS * 128)
    assert e == nw * epw and epw == nch * _C
    assert nch == ngrp * _GC and ngrp % _ES == 0 and _GC == 2 * _RING
    assert d % 16 == 0 and (npad // _NS) % _ZR == 0

    eidx = edge_index.reshape(2, nw, ngrp, _GC, _C)
    parts = _neighbor_partials(eidx, x, npad)

    bias = (b_neigh + b_self).reshape(1, d_out)
    bt = 1000

    def combine_body(p_ref, x_ref, wn_ref, ws_ref, b_ref, o_ref):
        neigh = p_ref[0] + p_ref[1]
        o_ref[...] = (
            lax.dot_general(neigh, wn_ref[...], (((1,), (1,)), ((), ())),
                            preferred_element_type=jnp.float32)
            + lax.dot_general(x_ref[...], ws_ref[...], (((1,), (1,)), ((), ())),
                              preferred_element_type=jnp.float32)
            + b_ref[...]
        )

    out = pl.pallas_call(
        combine_body,
        grid=(n // bt,),
        in_specs=[
            pl.BlockSpec((_NC, bt, d), lambda i: (0, i, 0)),
            pl.BlockSpec((bt, d), lambda i: (i, 0)),
            pl.BlockSpec((d_out, d), lambda i: (0, 0)),
            pl.BlockSpec((d_out, d), lambda i: (0, 0)),
            pl.BlockSpec((1, d_out), lambda i: (0, 0)),
        ],
        out_specs=pl.BlockSpec((bt, d_out), lambda i: (i, 0)),
        out_shape=jax.ShapeDtypeStruct((n, d_out), jnp.float32),
    )(parts, x, W_neigh, W_self, bias)
    return out
